# baseline (speedup 1.0000x reference)
import functools

import jax
import jax.numpy as jnp
from jax.experimental import pallas as pl
from jax.experimental.pallas import tpu as pltpu

_BATCH_BLK = 1024


def _mlp_kernel(x_ref, w1_ref, b1_ref, w2_ref, b2_ref, w3_ref, b3_ref, o_ref):
    x = x_ref[...]
    h = jnp.dot(x, w1_ref[...], preferred_element_type=jnp.float32)
    h = jnp.maximum(h + b1_ref[...], 0.0)
    h = jnp.dot(h, w2_ref[...], preferred_element_type=jnp.float32)
    h = jnp.maximum(h + b2_ref[...], 0.0)
    o = jnp.dot(h, w3_ref[...], preferred_element_type=jnp.float32)
    o_ref[...] = o + b3_ref[...]


@functools.partial(jax.jit, static_argnames=())
def kernel(modes_vec, W1, b1, W2, b2, W3, b3, type_n):
    del type_n
    batch, steps = modes_vec.shape
    blk = min(_BATCH_BLK, batch)
    grid = (batch // blk,)

    w1t = W1.T
    w2t = W2.T
    w3t = W3.T
    b1r = b1.reshape(1, -1)
    b2r = b2.reshape(1, -1)
    b3r = b3.reshape(1, -1)

    full = lambda i: (0, 0)
    out = pl.pallas_call(
        _mlp_kernel,
        grid=grid,
        in_specs=[
            pl.BlockSpec((blk, steps), lambda i: (i, 0)),
            pl.BlockSpec(w1t.shape, full),
            pl.BlockSpec(b1r.shape, full),
            pl.BlockSpec(w2t.shape, full),
            pl.BlockSpec(b2r.shape, full),
            pl.BlockSpec(w3t.shape, full),
            pl.BlockSpec(b3r.shape, full),
        ],
        out_specs=pl.BlockSpec((blk, 1), lambda i: (i, 0)),
        out_shape=jax.ShapeDtypeStruct((batch, 1), jnp.float32),
        compiler_params=pltpu.CompilerParams(
            dimension_semantics=("arbitrary",),
        ),
    )(modes_vec, w1t, b1r, w2t, b2r, w3t, b3r)
    return out


# speedup vs baseline: 1.2127x; 1.2127x over previous
import functools

import jax
import jax.numpy as jnp
from jax.experimental import pallas as pl
from jax.experimental.pallas import tpu as pltpu

_BATCH_BLK = 4096

_B1_ROW = 200
_W2_ROW = 208
_B2_ROW = 240
_W3_ROW = 241
_B3_ROW = 242


def _mlp_kernel(x_ref, p_ref, o_ref):
    x = x_ref[...]
    w1t = p_ref[0:200, :]
    h = jnp.dot(x, w1t, preferred_element_type=jnp.float32)
    h = jnp.maximum(h + p_ref[_B1_ROW:_B1_ROW + 1, :], 0.0)
    w2t = p_ref[_W2_ROW:_W2_ROW + 32, 0:8]
    z = jnp.dot(h, w2t, preferred_element_type=jnp.float32)
    z = z + p_ref[_B2_ROW:_B2_ROW + 1, 0:8]
    h2 = jnp.maximum(z, 0.0) * p_ref[_W3_ROW:_W3_ROW + 1, 0:8]
    o_ref[...] = jnp.sum(h2, axis=1, keepdims=True) + p_ref[_B3_ROW:_B3_ROW + 1, 0:1]


@functools.partial(jax.jit, static_argnames=())
def kernel(modes_vec, W1, b1, W2, b2, W3, b3, type_n):
    del type_n
    batch, steps = modes_vec.shape
    blk = min(_BATCH_BLK, batch)
    grid = (batch // blk,)

    p = jnp.zeros((256, 32), jnp.float32)
    p = p.at[0:steps, :].set(W1.T)
    p = p.at[_B1_ROW, :].set(b1)
    p = p.at[_W2_ROW:_W2_ROW + 32, 0:8].set(W2.T)
    p = p.at[_B2_ROW, 0:8].set(b2)
    p = p.at[_W3_ROW, 0:8].set(W3[0])
    p = p.at[_B3_ROW, 0].set(b3[0])

    out = pl.pallas_call(
        _mlp_kernel,
        grid=grid,
        in_specs=[
            pl.BlockSpec((blk, steps), lambda i: (i, 0)),
            pl.BlockSpec(p.shape, lambda i: (0, 0)),
        ],
        out_specs=pl.BlockSpec((blk, 1), lambda i: (i, 0)),
        out_shape=jax.ShapeDtypeStruct((batch, 1), jnp.float32),
        compiler_params=pltpu.CompilerParams(
            dimension_semantics=("arbitrary",),
        ),
    )(modes_vec, p)
    return out


# speedup vs baseline: 4.6548x; 3.8384x over previous
import functools

import jax
import jax.numpy as jnp
from jax.experimental import pallas as pl
from jax.experimental.pallas import tpu as pltpu

_LANE_BLK = 2048


def _mlp_kernel(x_ref, w1_ref, b1_ref, w2_ref, b2_ref, w3_ref, b3_ref, o_ref):
    x = x_ref[...]
    h = jax.lax.dot_general(
        w1_ref[...], x, (((1,), (0,)), ((), ())),
        preferred_element_type=jnp.float32)
    h = jnp.maximum(h + b1_ref[...].T, 0.0)
    z = jax.lax.dot_general(
        w2_ref[...], h, (((1,), (0,)), ((), ())),
        preferred_element_type=jnp.float32)
    h2 = jnp.maximum(z + b2_ref[...].T, 0.0) * w3_ref[...].T
    o_ref[...] = jnp.sum(h2, axis=0, keepdims=True) + b3_ref[...]


@functools.partial(jax.jit, static_argnames=())
def kernel(modes_vec, W1, b1, W2, b2, W3, b3, type_n):
    del type_n
    batch, steps = modes_vec.shape
    blk = min(_LANE_BLK, batch)
    grid = (batch // blk,)

    xt = modes_vec.T

    full = lambda i: (0, 0)
    outt = pl.pallas_call(
        _mlp_kernel,
        grid=grid,
        in_specs=[
            pl.BlockSpec((steps, blk), lambda i: (0, i)),
            pl.BlockSpec(W1.shape, full),
            pl.BlockSpec((1, W1.shape[0]), full),
            pl.BlockSpec(W2.shape, full),
            pl.BlockSpec((1, W2.shape[0]), full),
            pl.BlockSpec(W3.shape, full),
            pl.BlockSpec((1, 1), full),
        ],
        out_specs=pl.BlockSpec((1, blk), lambda i: (0, i)),
        out_shape=jax.ShapeDtypeStruct((1, batch), jnp.float32),
        compiler_params=pltpu.CompilerParams(
            dimension_semantics=("arbitrary",),
        ),
    )(xt, W1, b1.reshape(1, -1), W2, b2.reshape(1, -1), W3, b3.reshape(1, -1))
    return outt.reshape(batch, 1)


# speedup vs baseline: 4.6740x; 1.0041x over previous
import functools

import jax
import jax.numpy as jnp
from jax.experimental import pallas as pl
from jax.experimental.pallas import tpu as pltpu

_LANE_BLK = 2048


def _mlp_kernel(x_ref, w1_ref, b1_ref, w2_ref, b2_ref, w3_ref, b3_ref, o_ref):
    x = x_ref[...]
    h = jax.lax.dot_general(
        w1_ref[...], x, (((1,), (0,)), ((), ())),
        preferred_element_type=jnp.float32)
    h = jnp.maximum(h + b1_ref[...].T, 0.0)
    z = jax.lax.dot_general(
        w2_ref[...], h, (((1,), (0,)), ((), ())),
        preferred_element_type=jnp.float32)
    h2 = jnp.maximum(z + b2_ref[...].T, 0.0) * w3_ref[...].T
    o_ref[...] = jnp.sum(h2, axis=0) + b3_ref[0, 0]


@functools.partial(jax.jit, static_argnames=())
def kernel(modes_vec, W1, b1, W2, b2, W3, b3, type_n):
    del type_n
    batch, steps = modes_vec.shape
    blk = min(_LANE_BLK, batch)
    grid = (batch // blk,)

    xt = modes_vec.T

    full = lambda i: (0, 0)
    outt = pl.pallas_call(
        _mlp_kernel,
        grid=grid,
        in_specs=[
            pl.BlockSpec((steps, blk), lambda i: (0, i)),
            pl.BlockSpec(W1.shape, full),
            pl.BlockSpec((1, W1.shape[0]), full),
            pl.BlockSpec(W2.shape, full),
            pl.BlockSpec((1, W2.shape[0]), full),
            pl.BlockSpec(W3.shape, full),
            pl.BlockSpec((1, 1), full),
        ],
        out_specs=pl.BlockSpec((blk,), lambda i: (i,)),
        out_shape=jax.ShapeDtypeStruct((batch,), jnp.float32),
        compiler_params=pltpu.CompilerParams(
            dimension_semantics=("arbitrary",),
        ),
    )(xt, W1, b1.reshape(1, -1), W2, b2.reshape(1, -1), W3, b3.reshape(1, -1))
    return outt.reshape(batch, 1)


# speedup vs baseline: 6.0545x; 1.2954x over previous
import functools

import jax
import jax.numpy as jnp
from jax.experimental import pallas as pl
from jax.experimental.pallas import tpu as pltpu

_LANE_BLK = 4096


def _mlp_kernel(x_ref, w1_ref, b1_ref, w2_ref, b2_ref, w3_ref, b3_ref, o_ref):
    x = x_ref[...]
    h = jax.lax.dot_general(
        w1_ref[...], x, (((1,), (0,)), ((), ())),
        preferred_element_type=jnp.float32)
    h = jnp.maximum(h + b1_ref[...].T, 0.0)
    z = jax.lax.dot_general(
        w2_ref[...], h, (((1,), (0,)), ((), ())),
        preferred_element_type=jnp.float32)
    h2 = jnp.maximum(z + b2_ref[...].T, 0.0) * w3_ref[...].T
    o_ref[...] = jnp.sum(h2, axis=0) + b3_ref[0, 0]


@functools.partial(jax.jit, static_argnames=())
def kernel(modes_vec, W1, b1, W2, b2, W3, b3, type_n):
    del type_n
    batch, steps = modes_vec.shape
    blk = min(_LANE_BLK, batch)
    grid = (batch // blk,)

    xt = modes_vec.T

    full = lambda i: (0, 0)
    outt = pl.pallas_call(
        _mlp_kernel,
        grid=grid,
        in_specs=[
            pl.BlockSpec((steps, blk), lambda i: (0, i)),
            pl.BlockSpec(W1.shape, full),
            pl.BlockSpec((1, W1.shape[0]), full),
            pl.BlockSpec(W2.shape, full),
            pl.BlockSpec((1, W2.shape[0]), full),
            pl.BlockSpec(W3.shape, full),
            pl.BlockSpec((1, 1), full),
        ],
        out_specs=pl.BlockSpec((blk,), lambda i: (i,)),
        out_shape=jax.ShapeDtypeStruct((batch,), jnp.float32),
        compiler_params=pltpu.CompilerParams(
            dimension_semantics=("arbitrary",),
        ),
    )(xt, W1, b1.reshape(1, -1), W2, b2.reshape(1, -1), W3, b3.reshape(1, -1))
    return outt.reshape(batch, 1)


# speedup vs baseline: 6.4836x; 1.0709x over previous
import functools

import jax
import jax.numpy as jnp
from jax.experimental import pallas as pl
from jax.experimental.pallas import tpu as pltpu

_LANE_BLK = 8192


def _mlp_kernel(x_ref, w1_ref, b1_ref, w2_ref, b2_ref, w3_ref, b3_ref, o_ref):
    x = x_ref[...]
    h = jax.lax.dot_general(
        w1_ref[...], x, (((1,), (0,)), ((), ())),
        preferred_element_type=jnp.float32)
    h = jnp.maximum(h + b1_ref[...].T, 0.0)
    z = jax.lax.dot_general(
        w2_ref[...], h, (((1,), (0,)), ((), ())),
        preferred_element_type=jnp.float32)
    h2 = jnp.maximum(z + b2_ref[...].T, 0.0) * w3_ref[...].T
    o_ref[...] = jnp.sum(h2, axis=0) + b3_ref[0, 0]


@functools.partial(jax.jit, static_argnames=())
def kernel(modes_vec, W1, b1, W2, b2, W3, b3, type_n):
    del type_n
    batch, steps = modes_vec.shape
    blk = min(_LANE_BLK, batch)
    grid = (batch // blk,)

    xt = modes_vec.T

    full = lambda i: (0, 0)
    outt = pl.pallas_call(
        _mlp_kernel,
        grid=grid,
        in_specs=[
            pl.BlockSpec((steps, blk), lambda i: (0, i)),
            pl.BlockSpec(W1.shape, full),
            pl.BlockSpec((1, W1.shape[0]), full),
            pl.BlockSpec(W2.shape, full),
            pl.BlockSpec((1, W2.shape[0]), full),
            pl.BlockSpec(W3.shape, full),
            pl.BlockSpec((1, 1), full),
        ],
        out_specs=pl.BlockSpec((blk,), lambda i: (i,)),
        out_shape=jax.ShapeDtypeStruct((batch,), jnp.float32),
        compiler_params=pltpu.CompilerParams(
            dimension_semantics=("arbitrary",),
        ),
    )(xt, W1, b1.reshape(1, -1), W2, b2.reshape(1, -1), W3, b3.reshape(1, -1))
    return outt.reshape(batch, 1)
